# baseline (speedup 1.0000x reference)
import functools

import jax
import jax.numpy as jnp
from jax import lax
from jax.experimental import pallas as pl
from jax.experimental.pallas import tpu as pltpu
from jax.experimental.pallas import tpu_sc as plsc

_N_BINS = 10
_N = 4194304
_NC = 2
_NS = 16
_NW = _NC * _NS
_L = 16
_PER_W = _N // _NW
_CHUNK = 16384
_ROWS = 16
_ACC = _ROWS * _L


def _ece_body(logits_hbm, corr_hbm, bounds_hbm, out_conf, out_cnt,
              lbuf, cbuf, bvmem, acc_f, acc_q):
    wid = lax.axis_index("s") * _NC + lax.axis_index("c")
    base = wid * _PER_W

    pltpu.sync_copy(bounds_hbm, bvmem)
    zf = jnp.zeros((_L,), jnp.float32)
    zi = jnp.zeros((_L,), jnp.int32)
    for r in range(_ROWS):
        acc_f[pl.ds(r * _L, _L)] = zf
        acc_q[pl.ds(r * _L, _L)] = zi
    lane = lax.iota(jnp.int32, _L)
    bvec = bvmem[...]

    def vec_body(v, carry):
        off = v * _L
        l = lbuf[pl.ds(off, _L)]
        c = cbuf[pl.ds(off, _L)]
        j = (l * 10.0).astype(jnp.int32)
        glo = bvec.at[j].get(mode="promise_in_bounds")
        ghi = bvec.at[j + 1].get(mode="promise_in_bounds")
        row = (j + 1
               - jnp.where(l <= glo, 1, 0)
               + jnp.where(l > ghi, 1, 0))
        flat = row * _L + lane
        q = c * 16384 + 1
        plsc.addupdate_scatter(acc_f, [flat], l)
        plsc.addupdate_scatter(acc_q, [flat], q)
        return carry

    def chunk_body(ci, carry):
        start = base + ci * _CHUNK
        pltpu.sync_copy(logits_hbm.at[pl.ds(start, _CHUNK)], lbuf)
        pltpu.sync_copy(corr_hbm.at[pl.ds(start, _CHUNK)], cbuf)
        lax.fori_loop(0, _CHUNK // _L, vec_body, carry, unroll=4)
        return carry

    lax.fori_loop(0, _PER_W // _CHUNK, chunk_body, 0)

    pltpu.sync_copy(acc_f, out_conf.at[wid])
    pltpu.sync_copy(acc_q, out_cnt.at[wid])


@jax.jit
def _ece_sc(logits, corr, bounds):
    run = pl.kernel(
        _ece_body,
        out_type=(
            jax.ShapeDtypeStruct((_NW, _ACC), jnp.float32),
            jax.ShapeDtypeStruct((_NW, _ACC), jnp.int32),
        ),
        mesh=plsc.VectorSubcoreMesh(core_axis_name="c", subcore_axis_name="s"),
        compiler_params=pltpu.CompilerParams(needs_layout_passes=False),
        scratch_types=[
            pltpu.VMEM((_CHUNK,), jnp.float32),
            pltpu.VMEM((_CHUNK,), jnp.int32),
            pltpu.VMEM((_L,), jnp.float32),
            pltpu.VMEM((_ACC,), jnp.float32),
            pltpu.VMEM((_ACC,), jnp.int32),
        ],
    )
    return run(logits, corr, bounds)


def kernel(logits, correctness):
    corr = correctness.astype(jnp.int32)
    bounds = jnp.concatenate(
        [jnp.linspace(0.0, 1.0, _N_BINS + 1).astype(jnp.float32),
         jnp.full((5,), 2.0, jnp.float32)])
    conf_p, q_p = _ece_sc(logits, corr, bounds)

    conf_p = conf_p.reshape(_NW, _ROWS, _L)
    q_p = q_p.reshape(_NW, _ROWS, _L)
    sum_conf = jnp.sum(conf_p, axis=(0, 2))[1:_N_BINS + 1]
    cnt = jnp.sum((q_p & 16383).astype(jnp.float32), axis=(0, 2))[1:_N_BINS + 1]
    sum_acc = jnp.sum((q_p >> 14).astype(jnp.float32), axis=(0, 2))[1:_N_BINS + 1]

    total = jnp.float32(_N)
    prop = cnt / total
    safe = jnp.maximum(cnt, 1.0)
    acc_in = sum_acc / safe
    conf_in = sum_conf / safe
    contrib = jnp.abs(conf_in - acc_in) * prop
    ece = jnp.sum(jnp.where(cnt > 0, contrib, 0.0))
    return ece.reshape(1)
